# speedup vs baseline: 1.1386x; 1.1386x over previous
import functools

import jax
import jax.numpy as jnp
from jax import lax
from jax.experimental import pallas as pl
from jax.experimental.pallas import tpu as pltpu

K = 1024
THRESH = 20.0
KWTA = 8
T = 64
RED = 16384
BLK = 128
NBLK = K // BLK


def _fused_kernel(x_ref, w_ref, out_ref, spike_s, cnt_s, val_s):
    i = pl.program_id(0)
    out_blk = lax.dot_general(
        x_ref[...], w_ref[...], (((1,), (1,)), ((), ())),
        preferred_element_type=jnp.float32)
    pot = jnp.where(out_blk > THRESH, out_blk, 0.0)
    spike = jnp.where(out_blk > THRESH, 1.0, 0.0)
    cnt = jnp.sum(spike, axis=0, keepdims=True)
    first = jnp.clip((T - cnt).astype(jnp.int32), 0, T - 1)
    rows = lax.broadcasted_iota(jnp.int32, (T, BLK), 0)
    vals = jnp.sum(jnp.where(rows == first, pot, 0.0), axis=0,
                   keepdims=True)
    spike_s[:, pl.ds(i * BLK, BLK)] = spike
    cnt_s[:, pl.ds(i * BLK, BLK)] = cnt
    val_s[:, pl.ds(i * BLK, BLK)] = vals

    @pl.when(i == NBLK - 1)
    def _():
        cnt_all = cnt_s[...]
        val_all = val_s[...]
        v = jnp.max(val_all) * T
        total = cnt_all * (val_all + v)
        colid = lax.broadcasted_iota(jnp.int32, (1, K), 1)
        coef = jnp.zeros((1, K), jnp.float32)
        for _ in range(KWTA):
            m = jnp.max(total)
            idx = jnp.min(jnp.where(total == m, colid, K))
            sel = colid == idx
            coef = jnp.where(sel & (m != 0.0), 1.0, coef)
            total = jnp.where(sel, 0.0, total)
        out_ref[...] = spike_s[...] * coef


@jax.jit
def kernel(rec_field, W):
    x = rec_field.reshape(T, RED)
    w = W.reshape(K, RED)
    out = pl.pallas_call(
        _fused_kernel,
        grid=(NBLK,),
        in_specs=[
            pl.BlockSpec((T, RED), lambda i: (0, 0)),
            pl.BlockSpec((BLK, RED), lambda i: (i, 0)),
        ],
        out_specs=pl.BlockSpec((T, K), lambda i: (0, 0)),
        out_shape=jax.ShapeDtypeStruct((T, K), jnp.float32),
        scratch_shapes=[
            pltpu.VMEM((T, K), jnp.float32),
            pltpu.VMEM((1, K), jnp.float32),
            pltpu.VMEM((1, K), jnp.float32),
        ],
    )(x, w)
    return out.reshape(T, K, 1, 1)


# speedup vs baseline: 2.2365x; 1.9643x over previous
import functools

import jax
import jax.numpy as jnp
from jax import lax
from jax.experimental import pallas as pl
from jax.experimental.pallas import tpu as pltpu

K = 1024
THRESH = 20.0
KWTA = 8
T = 64
RED = 16384
RF = 64
LEN = 256
BLK = 128
NBLK = K // BLK


def _fused_kernel(x_ref, w_ref, out_ref, spike_s, cnt_s, val_s):
    i = pl.program_id(0)
    out_blk = jnp.zeros((T, BLK), jnp.float32)
    for r in range(RF):
        out_blk += lax.dot_general(
            x_ref[:, r, :], w_ref[:, r, :], (((1,), (1,)), ((), ())),
            preferred_element_type=jnp.float32)
    pot = jnp.where(out_blk > THRESH, out_blk, 0.0)
    spike = jnp.where(out_blk > THRESH, 1.0, 0.0)
    cnt = jnp.sum(spike, axis=0, keepdims=True)
    first = jnp.clip((T - cnt).astype(jnp.int32), 0, T - 1)
    rows = lax.broadcasted_iota(jnp.int32, (T, BLK), 0)
    vals = jnp.sum(jnp.where(rows == first, pot, 0.0), axis=0,
                   keepdims=True)
    spike_s[:, pl.ds(i * BLK, BLK)] = spike
    cnt_s[:, pl.ds(i * BLK, BLK)] = cnt
    val_s[:, pl.ds(i * BLK, BLK)] = vals

    @pl.when(i == NBLK - 1)
    def _():
        cnt_all = cnt_s[...]
        val_all = val_s[...]
        v = jnp.max(val_all) * T
        total = cnt_all * (val_all + v)
        colid = lax.broadcasted_iota(jnp.int32, (1, K), 1)
        coef = jnp.zeros((1, K), jnp.float32)
        for _ in range(KWTA):
            m = jnp.max(total)
            idx = jnp.min(jnp.where(total == m, colid, K))
            sel = colid == idx
            coef = jnp.where(sel & (m != 0.0), 1.0, coef)
            total = jnp.where(sel, 0.0, total)
        out_ref[...] = spike_s[...] * coef


@jax.jit
def kernel(rec_field, W):
    x = rec_field.reshape(T, RF, LEN)
    w = W.reshape(K, RF, LEN)
    out = pl.pallas_call(
        _fused_kernel,
        grid=(NBLK,),
        in_specs=[
            pl.BlockSpec((T, RF, LEN), lambda i: (0, 0, 0)),
            pl.BlockSpec((BLK, RF, LEN), lambda i: (i, 0, 0)),
        ],
        out_specs=pl.BlockSpec((T, K), lambda i: (0, 0)),
        out_shape=jax.ShapeDtypeStruct((T, K), jnp.float32),
        scratch_shapes=[
            pltpu.VMEM((T, K), jnp.float32),
            pltpu.VMEM((1, K), jnp.float32),
            pltpu.VMEM((1, K), jnp.float32),
        ],
    )(x, w)
    return out.reshape(T, K, 1, 1)
